# Initial kernel scaffold; baseline (speedup 1.0000x reference)
import jax
import jax.numpy as jnp
from jax.experimental import pallas as pl


def kernel(params, charges, crds_3d, atom_id, ring_id, hybr_id, arom_id, edge_index, batch, lgnd_id, slvn_id, rgnt_id, clst_id):
    raise NotImplementedError("write your pallas kernel here")


# speedup vs baseline: 1.7181x; 1.7181x over previous
import functools

import jax
import jax.numpy as jnp
from jax import lax
from jax.experimental import pallas as pl
from jax.experimental.pallas import tpu as pltpu
from jax.experimental.pallas import tpu_sc as plsc

XW = 80
FD = 64
MD = 16
H1 = 384
NW = 32
NC = 2
NS = 16


def _divisor_block(n, max_b, mult=8):
    best = None
    for b in range(mult, max_b + 1, mult):
        if n % b == 0:
            best = b
    if best is None:
        raise ValueError(f"no block for n={n} max={max_b}")
    return best


def _silu(x):
    return x * jax.nn.sigmoid(x)


def _ln(x, g, b, eps=1e-5):
    m = jnp.mean(x, axis=-1, keepdims=True)
    v = jnp.mean((x - m) * (x - m), axis=-1, keepdims=True)
    return (x - m) * jax.lax.rsqrt(v + eps) * g + b


def _pad_to(x, shape):
    pads = [(0, t - s) for s, t in zip(x.shape, shape)]
    return jnp.pad(x, pads)


def _embed_body(atom, ring, hybr, arom, nfeat, Ta, Tr, Th, Tar, Wc, b0,
                W2, b2, W3, b3, Sc, Sh, out):
    def oh(ref, k):
        ids = ref[...]
        i = lax.broadcasted_iota(jnp.int32, (ids.shape[0], k), 1)
        return (i == ids).astype(jnp.float32)

    nf = nfeat[...]
    h = (jnp.dot(oh(atom, 16), Ta[...]) + jnp.dot(oh(ring, 8), Tr[...])
         + jnp.dot(oh(hybr, 8), Th[...]) + jnp.dot(oh(arom, 8), Tar[...])
         + jnp.dot(nf, Wc[...]) + b0[...])
    h = _silu(h)
    h = _silu(jnp.dot(h, W2[...]) + b2[...])
    h = _silu(jnp.dot(h, W3[...]) + b3[...])
    out[...] = jnp.dot(nf, Sc[...]) + jnp.dot(h, Sh[...])


def _edge_body(gd, gs, W1d, W1s, wdv, b1, W2, b2, g, b, out):
    gdv = gd[...]
    gsv = gs[...]
    d = gsv - gdv
    cmask = (lax.broadcasted_iota(jnp.int32, (1, XW), 1) < 3).astype(jnp.float32)
    rd = jnp.sum(d * d * cmask, axis=1, keepdims=True)
    t = (jnp.dot(gdv, W1d[...]) + jnp.dot(gsv, W1s[...])
         + rd * wdv[...] + b1[...])
    t = _silu(t)
    m = _silu(jnp.dot(t, W2[...]) + b2[...])
    out[...] = _ln(m, g[...], b[...])


def _node_body(xc, p0, p1, g2, b2, gn, bn, n1h, n1m, bn1, Wn2, bn2,
               Ssel, Sh, out):
    x = xc[...]
    mi = _ln(p0[...] + p1[...], g2[...], b2[...])
    feats = jnp.dot(x, Ssel[...])
    h = _ln(feats, gn[...], bn[...])
    u = _silu(jnp.dot(h, n1h[...]) + jnp.dot(mi, n1m[...]) + bn1[...])
    hnew = feats + jnp.dot(u, Wn2[...]) + bn2[...]
    cmask = (lax.broadcasted_iota(jnp.int32, (1, XW), 1) < 3).astype(jnp.float32)
    out[...] = x * cmask + jnp.dot(hnew, Sh[...])


def _post_body(x1, x2, x3, P1, P2, P3, bp1, W2, bp2, W3, bp3, out):
    f = (jnp.dot(x1[...], P1[...]) + jnp.dot(x2[...], P2[...])
         + jnp.dot(x3[...], P3[...]) + bp1[...])
    f = _silu(f)
    f = _silu(jnp.dot(f, W2[...]) + bp2[...])
    out[...] = _silu(jnp.dot(f, W3[...]) + bp3[...])


def _final_body(p0, p1, lg, sl, rg, cl, Wp, TL, TS, TR, TCc, b1,
                W2, b2, W3, b3, W4, b4, out):
    def oh(ref, k):
        ids = ref[...]
        i = lax.broadcasted_iota(jnp.int32, (ids.shape[0], k), 1)
        return (i == ids).astype(jnp.float32)

    z = (jnp.dot(p0[...] + p1[...], Wp[...]) + jnp.dot(oh(lg, 16), TL[...])
         + jnp.dot(oh(sl, 16), TS[...]) + jnp.dot(oh(rg, 8), TR[...])
         + jnp.dot(oh(cl, 8), TCc[...]) + b1[...])
    z = _silu(z)
    z = _silu(jnp.dot(z, W2[...]) + b2[...])
    z = _silu(jnp.dot(z, W3[...]) + b3[...])
    out[...] = jnp.dot(z, W4[...]) + b4[...]


def _tc_call(body, grid, blocked, full, out_block, out_shape):
    full = [a.reshape(1, -1) if a.ndim == 1 else a for a in full]
    in_specs = [pl.BlockSpec(bs, lambda i: (i, 0)) for _, bs in blocked]
    in_specs += [pl.BlockSpec(a.shape, lambda i, _r=len(a.shape): (0,) * _r)
                 for a in full]
    return pl.pallas_call(
        body,
        grid=(grid,),
        in_specs=in_specs,
        out_specs=pl.BlockSpec(out_block, lambda i: (i, 0)),
        out_shape=out_shape,
    )(*[a for a, _ in blocked], *full)


def _sc_gather(xcat, src, dst, E):
    CH = _divisor_block(E // NW, 800)
    NIT = E // NW // CH
    mesh = plsc.VectorSubcoreMesh(core_axis_name="c", subcore_axis_name="s")

    @functools.partial(
        pl.kernel, mesh=mesh,
        out_type=(jax.ShapeDtypeStruct((E, XW), jnp.float32),
                  jax.ShapeDtypeStruct((E, XW), jnp.float32)),
        scratch_types=[pltpu.VMEM((CH,), jnp.int32),
                       pltpu.VMEM((CH, XW), jnp.float32),
                       pltpu.SemaphoreType.DMA],
        compiler_params=pltpu.CompilerParams(use_tc_tiling_on_sc=False),
    )
    def k(x_hbm, s_hbm, d_hbm, gs_hbm, gd_hbm, idx_v, rows_v, sem):
        wid = lax.axis_index("s") * NC + lax.axis_index("c")
        base = wid * (E // NW)

        def step(i, _):
            off = base + i * CH
            pltpu.sync_copy(s_hbm.at[pl.ds(off, CH)], idx_v)
            pltpu.async_copy(x_hbm.at[idx_v], rows_v, sem).wait()
            pltpu.sync_copy(rows_v, gs_hbm.at[pl.ds(off, CH)])
            pltpu.sync_copy(d_hbm.at[pl.ds(off, CH)], idx_v)
            pltpu.async_copy(x_hbm.at[idx_v], rows_v, sem).wait()
            pltpu.sync_copy(rows_v, gd_hbm.at[pl.ds(off, CH)])
            return _

        lax.fori_loop(0, NIT, step, None)

    return k(xcat, src, dst)


def _sc_scatter(vals, idx, zeros, n_rows, width, max_ch):
    R = vals.shape[0]
    PW = R // NW
    CH = _divisor_block(PW, max_ch)
    NIT = PW // CH
    STR = n_rows // NS
    mesh = plsc.VectorSubcoreMesh(core_axis_name="c", subcore_axis_name="s")

    @functools.partial(
        pl.kernel, mesh=mesh,
        out_type=jax.ShapeDtypeStruct((NC, n_rows, width), jnp.float32),
        scratch_types=[pltpu.VMEM_SHARED((n_rows, width), jnp.float32),
                       pltpu.VMEM((CH,), jnp.int32),
                       pltpu.VMEM((CH, width), jnp.float32)],
        compiler_params=pltpu.CompilerParams(use_tc_tiling_on_sc=False),
    )
    def k(v_hbm, i_hbm, z_hbm, out_hbm, accum, idx_v, rows_v):
        c = lax.axis_index("c")
        s = lax.axis_index("s")
        wid = s * NC + c
        pltpu.sync_copy(z_hbm.at[pl.ds(s * STR, STR)],
                        accum.at[pl.ds(s * STR, STR)])
        plsc.subcore_barrier()
        base = wid * PW

        def step(i, _):
            off = base + i * CH
            pltpu.sync_copy(i_hbm.at[pl.ds(off, CH)], idx_v)
            pltpu.sync_copy(v_hbm.at[pl.ds(off, CH)], rows_v)
            pltpu.sync_copy(rows_v, accum.at[idx_v], add=True)
            return _

        lax.fori_loop(0, NIT, step, None)
        plsc.subcore_barrier()
        pltpu.sync_copy(accum.at[pl.ds(s * STR, STR)],
                        out_hbm.at[c, pl.ds(s * STR, STR)])

    return k(vals, idx, zeros)


def kernel(params, charges, crds_3d, atom_id, ring_id, hybr_id, arom_id,
           edge_index, batch, lgnd_id, slvn_id, rgnt_id, clst_id):
    N = charges.shape[0]
    E = edge_index.shape[1]
    G = lgnd_id.shape[0]
    NP = -(-N // (NW * 8)) * (NW * 8)
    EP = -(-E // (NW * 800)) * (NW * 800)
    NSEG = -(-(G + 1) // 128) * 128
    BN = _divisor_block(NP, 2048)
    BE = _divisor_block(EP, 4096)
    f32 = jnp.float32

    p = params
    eye = jnp.eye(FD, dtype=f32)
    Ssel = jnp.zeros((XW, FD), f32).at[3:3 + FD].set(eye)
    Sh = Ssel.T
    Sc = jnp.zeros((8, XW), f32).at[0:3, 0:3].set(jnp.eye(3, dtype=f32))

    def padi(x):
        return jnp.pad(x.astype(jnp.int32), (0, NP - N)).reshape(NP, 1)

    nfeat = jnp.zeros((NP, 8), f32)
    nfeat = nfeat.at[:N, 0:3].set(crds_3d).at[:N, 3].set(charges[:, 0])
    atom_p, ring_p, hybr_p, arom_p = map(padi, (atom_id, ring_id, hybr_id, arom_id))
    src = jnp.pad(edge_index[0].astype(jnp.int32), (0, EP - E))
    dst = jnp.pad(edge_index[1].astype(jnp.int32), (0, EP - E),
                  constant_values=N)
    batch_p = jnp.pad(batch.astype(jnp.int32), (0, NP - N), constant_values=G)
    z16 = jnp.zeros((NP, MD), f32)
    z128 = jnp.zeros((NSEG, 2 * FD), f32)

    w1 = p["pre1"]["w"]
    Ta = _pad_to(p["atom_em"] @ w1[0:64], (16, 128))
    Tr = _pad_to(p["ring_em"] @ w1[64:128], (8, 128))
    Th = _pad_to(p["hybr_em"] @ w1[128:192], (8, 128))
    Tar = _pad_to(p["arom_em"] @ w1[192:256], (8, 128))
    cw = p["chrg"]["w"] @ w1[256:320]
    Wc = jnp.zeros((8, 128), f32).at[3].set(cw[0])
    b0 = p["pre1"]["b"] + p["chrg"]["b"] @ w1[256:320]

    xcat = _tc_call(
        _embed_body, NP // BN,
        [(atom_p, (BN, 1)), (ring_p, (BN, 1)), (hybr_p, (BN, 1)),
         (arom_p, (BN, 1)), (nfeat, (BN, 8))],
        [Ta, Tr, Th, Tar, Wc, b0, p["pre2"]["w"], p["pre2"]["b"],
         p["pre3"]["w"], p["pre3"]["b"], Sc, Sh],
        (BN, XW), jax.ShapeDtypeStruct((NP, XW), f32))

    xcats = []
    for kp in p["kernels"]:
        e1w, e1b = kp["e1"]["w"], kp["e1"]["b"]
        W1d = jnp.zeros((XW, H1), f32).at[3:3 + FD, 0:258].set(e1w[0:64])
        W1s = jnp.zeros((XW, H1), f32).at[3:3 + FD, 0:258].set(e1w[64:128])
        wdv = _pad_to(e1w[128:129], (1, H1))
        b1 = _pad_to(e1b, (H1,))
        W2 = _pad_to(kp["e2"]["w"], (H1, MD))

        gs, gd = _sc_gather(xcat, src, dst, EP)
        m = _tc_call(
            _edge_body, EP // BE,
            [(gd, (BE, XW)), (gs, (BE, XW))],
            [W1d, W1s, wdv, b1, W2, kp["e2"]["b"], kp["en1_g"], kp["en1_b"]],
            (BE, MD), jax.ShapeDtypeStruct((EP, MD), f32))

        parts = _sc_scatter(m, dst, z16, NP, MD, 800)
        xcat = _tc_call(
            _node_body, NP // BN,
            [(xcat, (BN, XW)), (parts[0], (BN, MD)), (parts[1], (BN, MD))],
            [kp["en2_g"], kp["en2_b"], kp["nn1_g"], kp["nn1_b"],
             kp["n1"]["w"][0:FD], kp["n1"]["w"][FD:FD + MD], kp["n1"]["b"],
             kp["n2"]["w"], kp["n2"]["b"], Ssel, Sh],
            (BN, XW), jax.ShapeDtypeStruct((NP, XW), f32))
        xcats.append(xcat)

    pw = p["post1"]["w"]
    Ps = [jnp.zeros((XW, 128), f32).at[3:3 + FD].set(pw[64 * l:64 * (l + 1)])
          for l in range(3)]
    f = _tc_call(
        _post_body, NP // BN,
        [(xcats[0], (BN, XW)), (xcats[1], (BN, XW)), (xcats[2], (BN, XW))],
        [Ps[0], Ps[1], Ps[2], p["post1"]["b"], p["post2"]["w"], p["post2"]["b"],
         p["post3"]["w"], p["post3"]["b"]],
        (BN, 128), jax.ShapeDtypeStruct((NP, 128), f32))

    pooled = _sc_scatter(f, batch_p, z128, NSEG, 2 * FD, 784)

    wp1 = p["pp1"]["w"]
    Wp = wp1[0:128]
    TL = _pad_to(p["lig_emb"] @ wp1[128:192], (16, 512))
    TS = _pad_to(p["sol_emb"] @ wp1[192:256], (16, 512))
    TR = _pad_to(p["rgn_emb"] @ wp1[256:320], (8, 512))
    TCc = _pad_to(p["cat_emb"] @ wp1[320:384], (8, 512))

    def padg(x):
        return jnp.pad(x.astype(jnp.int32), (0, NSEG - G)).reshape(NSEG, 1)

    out = _tc_call(
        _final_body, 1,
        [(pooled[0], (NSEG, 128)), (pooled[1], (NSEG, 128)),
         (padg(lgnd_id), (NSEG, 1)), (padg(slvn_id), (NSEG, 1)),
         (padg(rgnt_id), (NSEG, 1)), (padg(clst_id), (NSEG, 1))],
        [Wp, TL, TS, TR, TCc, p["pp1"]["b"], p["pp2"]["w"], p["pp2"]["b"],
         p["pp3"]["w"], p["pp3"]["b"], p["pp4"]["w"], p["pp4"]["b"]],
        (NSEG, 1), jax.ShapeDtypeStruct((NSEG, 1), f32))
    return out[:G, 0]


# speedup vs baseline: 2.0283x; 1.1805x over previous
import functools

import jax
import jax.numpy as jnp
from jax import lax
from jax.experimental import pallas as pl
from jax.experimental.pallas import tpu as pltpu
from jax.experimental.pallas import tpu_sc as plsc

XW = 80
FD = 64
MD = 16
H1 = 384
NW = 32
NC = 2
NS = 16


def _divisor_block(n, max_b, mult=8):
    best = None
    for b in range(mult, max_b + 1, mult):
        if n % b == 0:
            best = b
    if best is None:
        raise ValueError(f"no block for n={n} max={max_b}")
    return best


def _silu(x):
    return x * jax.nn.sigmoid(x)


def _ln(x, g, b, eps=1e-5):
    m = jnp.mean(x, axis=-1, keepdims=True)
    v = jnp.mean((x - m) * (x - m), axis=-1, keepdims=True)
    return (x - m) * jax.lax.rsqrt(v + eps) * g + b


def _pad_to(x, shape):
    pads = [(0, t - s) for s, t in zip(x.shape, shape)]
    return jnp.pad(x, pads)


def _embed_body(atom, ring, hybr, arom, nfeat, Ta, Tr, Th, Tar, Wc, b0,
                W2, b2, W3, b3, Sc, Sh, out):
    def oh(ref, k):
        ids = ref[...]
        i = lax.broadcasted_iota(jnp.int32, (ids.shape[0], k), 1)
        return (i == ids).astype(jnp.float32)

    nf = nfeat[...]
    h = (jnp.dot(oh(atom, 16), Ta[...]) + jnp.dot(oh(ring, 8), Tr[...])
         + jnp.dot(oh(hybr, 8), Th[...]) + jnp.dot(oh(arom, 8), Tar[...])
         + jnp.dot(nf, Wc[...]) + b0[...])
    h = _silu(h)
    h = _silu(jnp.dot(h, W2[...]) + b2[...])
    h = _silu(jnp.dot(h, W3[...]) + b3[...])
    out[...] = jnp.dot(nf, Sc[...]) + jnp.dot(h, Sh[...])


def _edge_body(gd, gs, W1d, W1s, wdv, b1, W2, b2, g, b, out):
    gdv = gd[...]
    gsv = gs[...]
    d = gsv - gdv
    cmask = (lax.broadcasted_iota(jnp.int32, (1, XW), 1) < 3).astype(jnp.float32)
    rd = jnp.sum(d * d * cmask, axis=1, keepdims=True)
    t = (jnp.dot(gdv.astype(jnp.bfloat16), W1d[...],
                 preferred_element_type=jnp.float32)
         + jnp.dot(gsv.astype(jnp.bfloat16), W1s[...],
                   preferred_element_type=jnp.float32)
         + rd * wdv[...] + b1[...])
    t = _silu(t)
    m = _silu(jnp.dot(t.astype(jnp.bfloat16), W2[...],
                      preferred_element_type=jnp.float32) + b2[...])
    out[...] = _ln(m, g[...], b[...])


def _node_body(xc, p0, p1, g2, b2, gn, bn, n1h, n1m, bn1, Wn2, bn2,
               Ssel, Sh, out):
    x = xc[...]
    mi = _ln(p0[...] + p1[...], g2[...], b2[...])
    feats = jnp.dot(x, Ssel[...])
    h = _ln(feats, gn[...], bn[...])
    u = _silu(jnp.dot(h, n1h[...]) + jnp.dot(mi, n1m[...]) + bn1[...])
    hnew = feats + jnp.dot(u, Wn2[...]) + bn2[...]
    cmask = (lax.broadcasted_iota(jnp.int32, (1, XW), 1) < 3).astype(jnp.float32)
    out[...] = x * cmask + jnp.dot(hnew, Sh[...])


def _post_body(x1, x2, x3, P1, P2, P3, bp1, W2, bp2, W3, bp3, out):
    f = (jnp.dot(x1[...], P1[...]) + jnp.dot(x2[...], P2[...])
         + jnp.dot(x3[...], P3[...]) + bp1[...])
    f = _silu(f)
    f = _silu(jnp.dot(f, W2[...]) + bp2[...])
    out[...] = _silu(jnp.dot(f, W3[...]) + bp3[...])


def _final_body(p0, p1, lg, sl, rg, cl, Wp, TL, TS, TR, TCc, b1,
                W2, b2, W3, b3, W4, b4, out):
    def oh(ref, k):
        ids = ref[...]
        i = lax.broadcasted_iota(jnp.int32, (ids.shape[0], k), 1)
        return (i == ids).astype(jnp.float32)

    z = (jnp.dot(p0[...] + p1[...], Wp[...]) + jnp.dot(oh(lg, 16), TL[...])
         + jnp.dot(oh(sl, 16), TS[...]) + jnp.dot(oh(rg, 8), TR[...])
         + jnp.dot(oh(cl, 8), TCc[...]) + b1[...])
    z = _silu(z)
    z = _silu(jnp.dot(z, W2[...]) + b2[...])
    z = _silu(jnp.dot(z, W3[...]) + b3[...])
    out[...] = jnp.dot(z, W4[...]) + b4[...]


def _tc_call(body, grid, blocked, full, out_block, out_shape):
    full = [a.reshape(1, -1) if a.ndim == 1 else a for a in full]
    in_specs = [pl.BlockSpec(bs, lambda i: (i, 0)) for _, bs in blocked]
    in_specs += [pl.BlockSpec(a.shape, lambda i, _r=len(a.shape): (0,) * _r)
                 for a in full]
    return pl.pallas_call(
        body,
        grid=(grid,),
        in_specs=in_specs,
        out_specs=pl.BlockSpec(out_block, lambda i: (i, 0)),
        out_shape=out_shape,
    )(*[a for a, _ in blocked], *full)


def _sc_gather(xcat, src, dst, E):
    CH = _divisor_block(E // NW, 800)
    NIT = E // NW // CH
    mesh = plsc.VectorSubcoreMesh(core_axis_name="c", subcore_axis_name="s")

    CH2 = CH // 2

    @functools.partial(
        pl.kernel, mesh=mesh,
        out_type=(jax.ShapeDtypeStruct((E, XW), jnp.float32),
                  jax.ShapeDtypeStruct((E, XW), jnp.float32)),
        scratch_types=[pltpu.VMEM((CH2,), jnp.int32),
                       pltpu.VMEM((CH2,), jnp.int32),
                       pltpu.VMEM((CH2, XW), jnp.float32),
                       pltpu.VMEM((CH2, XW), jnp.float32),
                       pltpu.SemaphoreType.DMA,
                       pltpu.SemaphoreType.DMA],
        compiler_params=pltpu.CompilerParams(use_tc_tiling_on_sc=False),
    )
    def k(x_hbm, s_hbm, d_hbm, gs_hbm, gd_hbm, idx_s, idx_d, rows_s, rows_d,
          sem_s, sem_d):
        wid = lax.axis_index("s") * NC + lax.axis_index("c")
        base = wid * (E // NW)

        def step(i, _):
            off = base + i * CH2
            pltpu.sync_copy(s_hbm.at[pl.ds(off, CH2)], idx_s)
            cp_s = pltpu.async_copy(x_hbm.at[idx_s], rows_s, sem_s)
            pltpu.sync_copy(d_hbm.at[pl.ds(off, CH2)], idx_d)
            cp_d = pltpu.async_copy(x_hbm.at[idx_d], rows_d, sem_d)
            cp_s.wait()
            pltpu.sync_copy(rows_s, gs_hbm.at[pl.ds(off, CH2)])
            cp_d.wait()
            pltpu.sync_copy(rows_d, gd_hbm.at[pl.ds(off, CH2)])
            return _

        lax.fori_loop(0, 2 * NIT, step, None)

    return k(xcat, src, dst)


def _sc_scatter(vals, idx, zeros, n_rows, width, max_ch):
    R = vals.shape[0]
    PW = R // NW
    CH = _divisor_block(PW, max_ch)
    NIT = PW // CH
    STR = n_rows // NS
    mesh = plsc.VectorSubcoreMesh(core_axis_name="c", subcore_axis_name="s")

    @functools.partial(
        pl.kernel, mesh=mesh,
        out_type=jax.ShapeDtypeStruct((NC, n_rows, width), jnp.float32),
        scratch_types=[pltpu.VMEM_SHARED((n_rows, width), jnp.float32),
                       pltpu.VMEM((CH,), jnp.int32),
                       pltpu.VMEM((CH, width), jnp.float32)],
        compiler_params=pltpu.CompilerParams(use_tc_tiling_on_sc=False),
    )
    def k(v_hbm, i_hbm, z_hbm, out_hbm, accum, idx_v, rows_v):
        c = lax.axis_index("c")
        s = lax.axis_index("s")
        wid = s * NC + c
        pltpu.sync_copy(z_hbm.at[pl.ds(s * STR, STR)],
                        accum.at[pl.ds(s * STR, STR)])
        plsc.subcore_barrier()
        base = wid * PW

        def step(i, _):
            off = base + i * CH
            pltpu.sync_copy(i_hbm.at[pl.ds(off, CH)], idx_v)
            pltpu.sync_copy(v_hbm.at[pl.ds(off, CH)], rows_v)
            pltpu.sync_copy(rows_v, accum.at[idx_v], add=True)
            return _

        lax.fori_loop(0, NIT, step, None)
        plsc.subcore_barrier()
        pltpu.sync_copy(accum.at[pl.ds(s * STR, STR)],
                        out_hbm.at[c, pl.ds(s * STR, STR)])

    return k(vals, idx, zeros)


def kernel(params, charges, crds_3d, atom_id, ring_id, hybr_id, arom_id,
           edge_index, batch, lgnd_id, slvn_id, rgnt_id, clst_id):
    N = charges.shape[0]
    E = edge_index.shape[1]
    G = lgnd_id.shape[0]
    NP = -(-N // (NW * 8)) * (NW * 8)
    EP = -(-E // (NW * 800)) * (NW * 800)
    NSEG = -(-(G + 1) // 128) * 128
    BN = _divisor_block(NP, 2048)
    BE = _divisor_block(EP, 4096)
    f32 = jnp.float32

    p = params
    eye = jnp.eye(FD, dtype=f32)
    Ssel = jnp.zeros((XW, FD), f32).at[3:3 + FD].set(eye)
    Sh = Ssel.T
    Sc = jnp.zeros((8, XW), f32).at[0:3, 0:3].set(jnp.eye(3, dtype=f32))

    def padi(x):
        return jnp.pad(x.astype(jnp.int32), (0, NP - N)).reshape(NP, 1)

    nfeat = jnp.zeros((NP, 8), f32)
    nfeat = nfeat.at[:N, 0:3].set(crds_3d).at[:N, 3].set(charges[:, 0])
    atom_p, ring_p, hybr_p, arom_p = map(padi, (atom_id, ring_id, hybr_id, arom_id))
    src = jnp.pad(edge_index[0].astype(jnp.int32), (0, EP - E))
    dst = jnp.pad(edge_index[1].astype(jnp.int32), (0, EP - E),
                  constant_values=N)
    batch_p = jnp.pad(batch.astype(jnp.int32), (0, NP - N), constant_values=G)
    z16 = jnp.zeros((NP, MD), f32)
    z128 = jnp.zeros((NSEG, 2 * FD), f32)

    w1 = p["pre1"]["w"]
    Ta = _pad_to(p["atom_em"] @ w1[0:64], (16, 128))
    Tr = _pad_to(p["ring_em"] @ w1[64:128], (8, 128))
    Th = _pad_to(p["hybr_em"] @ w1[128:192], (8, 128))
    Tar = _pad_to(p["arom_em"] @ w1[192:256], (8, 128))
    cw = p["chrg"]["w"] @ w1[256:320]
    Wc = jnp.zeros((8, 128), f32).at[3].set(cw[0])
    b0 = p["pre1"]["b"] + p["chrg"]["b"] @ w1[256:320]

    xcat = _tc_call(
        _embed_body, NP // BN,
        [(atom_p, (BN, 1)), (ring_p, (BN, 1)), (hybr_p, (BN, 1)),
         (arom_p, (BN, 1)), (nfeat, (BN, 8))],
        [Ta, Tr, Th, Tar, Wc, b0, p["pre2"]["w"], p["pre2"]["b"],
         p["pre3"]["w"], p["pre3"]["b"], Sc, Sh],
        (BN, XW), jax.ShapeDtypeStruct((NP, XW), f32))

    xcats = []
    for kp in p["kernels"]:
        e1w, e1b = kp["e1"]["w"], kp["e1"]["b"]
        W1d = jnp.zeros((XW, H1), f32).at[3:3 + FD, 0:258].set(e1w[0:64])
        W1s = jnp.zeros((XW, H1), f32).at[3:3 + FD, 0:258].set(e1w[64:128])
        wdv = _pad_to(e1w[128:129], (1, H1))
        b1 = _pad_to(e1b, (H1,))
        W2 = _pad_to(kp["e2"]["w"], (H1, MD))

        gs, gd = _sc_gather(xcat, src, dst, EP)
        m = _tc_call(
            _edge_body, EP // BE,
            [(gd, (BE, XW)), (gs, (BE, XW))],
            [W1d.astype(jnp.bfloat16), W1s.astype(jnp.bfloat16), wdv, b1,
             W2.astype(jnp.bfloat16), kp["e2"]["b"], kp["en1_g"], kp["en1_b"]],
            (BE, MD), jax.ShapeDtypeStruct((EP, MD), f32))

        parts = _sc_scatter(m, dst, z16, NP, MD, 800)
        xcat = _tc_call(
            _node_body, NP // BN,
            [(xcat, (BN, XW)), (parts[0], (BN, MD)), (parts[1], (BN, MD))],
            [kp["en2_g"], kp["en2_b"], kp["nn1_g"], kp["nn1_b"],
             kp["n1"]["w"][0:FD], kp["n1"]["w"][FD:FD + MD], kp["n1"]["b"],
             kp["n2"]["w"], kp["n2"]["b"], Ssel, Sh],
            (BN, XW), jax.ShapeDtypeStruct((NP, XW), f32))
        xcats.append(xcat)

    pw = p["post1"]["w"]
    Ps = [jnp.zeros((XW, 128), f32).at[3:3 + FD].set(pw[64 * l:64 * (l + 1)])
          for l in range(3)]
    f = _tc_call(
        _post_body, NP // BN,
        [(xcats[0], (BN, XW)), (xcats[1], (BN, XW)), (xcats[2], (BN, XW))],
        [Ps[0], Ps[1], Ps[2], p["post1"]["b"], p["post2"]["w"], p["post2"]["b"],
         p["post3"]["w"], p["post3"]["b"]],
        (BN, 128), jax.ShapeDtypeStruct((NP, 128), f32))

    pooled = _sc_scatter(f, batch_p, z128, NSEG, 2 * FD, 784)

    wp1 = p["pp1"]["w"]
    Wp = wp1[0:128]
    TL = _pad_to(p["lig_emb"] @ wp1[128:192], (16, 512))
    TS = _pad_to(p["sol_emb"] @ wp1[192:256], (16, 512))
    TR = _pad_to(p["rgn_emb"] @ wp1[256:320], (8, 512))
    TCc = _pad_to(p["cat_emb"] @ wp1[320:384], (8, 512))

    def padg(x):
        return jnp.pad(x.astype(jnp.int32), (0, NSEG - G)).reshape(NSEG, 1)

    out = _tc_call(
        _final_body, 1,
        [(pooled[0], (NSEG, 128)), (pooled[1], (NSEG, 128)),
         (padg(lgnd_id), (NSEG, 1)), (padg(slvn_id), (NSEG, 1)),
         (padg(rgnt_id), (NSEG, 1)), (padg(clst_id), (NSEG, 1))],
        [Wp, TL, TS, TR, TCc, p["pp1"]["b"], p["pp2"]["w"], p["pp2"]["b"],
         p["pp3"]["w"], p["pp3"]["b"], p["pp4"]["w"], p["pp4"]["b"]],
        (NSEG, 1), jax.ShapeDtypeStruct((NSEG, 1), f32))
    return out[:G, 0]


# speedup vs baseline: 2.1344x; 1.0523x over previous
import functools

import jax
import jax.numpy as jnp
from jax import lax
from jax.experimental import pallas as pl
from jax.experimental.pallas import tpu as pltpu
from jax.experimental.pallas import tpu_sc as plsc

XW = 128
FL = 8
FD = 64
MD = 16
H1 = 384
NW = 32
NC = 2
NS = 16


def _divisor_block(n, max_b, mult=8):
    best = None
    for b in range(mult, max_b + 1, mult):
        if n % b == 0:
            best = b
    if best is None:
        raise ValueError(f"no block for n={n} max={max_b}")
    return best


def _silu(x):
    return x * jax.nn.sigmoid(x)


def _ln(x, g, b, eps=1e-5):
    m = jnp.mean(x, axis=-1, keepdims=True)
    v = jnp.mean((x - m) * (x - m), axis=-1, keepdims=True)
    return (x - m) * jax.lax.rsqrt(v + eps) * g + b


def _pad_to(x, shape):
    pads = [(0, t - s) for s, t in zip(x.shape, shape)]
    return jnp.pad(x, pads)


def _coords_to_row(c, n_rows):
    return jnp.concatenate(
        [c, jnp.zeros((n_rows, XW - 3), jnp.float32)], axis=1)


def _feats_to_row(h, n_rows):
    return jnp.concatenate(
        [jnp.zeros((n_rows, FL), jnp.float32), h,
         jnp.zeros((n_rows, XW - FL - FD), jnp.float32)], axis=1)


def _embed_body(atom, ring, hybr, arom, nfeat, Ta, Tr, Th, Tar, Wc, b0,
                W2, b2, W3, b3, xout, fout):
    def oh(ref, k):
        ids = ref[...]
        i = lax.broadcasted_iota(jnp.int32, (ids.shape[0], k), 1)
        return (i == ids).astype(jnp.float32)

    nf = nfeat[...]
    B = nf.shape[0]
    h = (jnp.dot(oh(atom, 16), Ta[...]) + jnp.dot(oh(ring, 8), Tr[...])
         + jnp.dot(oh(hybr, 8), Th[...]) + jnp.dot(oh(arom, 8), Tar[...])
         + jnp.dot(nf, Wc[...]) + b0[...])
    h = _silu(h)
    h = _silu(jnp.dot(h, W2[...]) + b2[...])
    h = _silu(jnp.dot(h, W3[...]) + b3[...])
    fout[...] = h
    xout[...] = _coords_to_row(nf[:, 0:3], B) + _feats_to_row(h, B)


def _edge_body(gd, gs, W1d, W1s, wdv, b1, W2, b2, g, b, out):
    gdv = gd[...]
    gsv = gs[...]
    B = gdv.shape[0]
    d = gsv[:, 0:3] - gdv[:, 0:3]
    rd = jnp.sum(d * d, axis=1, keepdims=True)
    t = (jnp.dot(gdv.astype(jnp.bfloat16), W1d[...],
                 preferred_element_type=jnp.float32)
         + jnp.dot(gsv.astype(jnp.bfloat16), W1s[...],
                   preferred_element_type=jnp.float32)
         + rd * wdv[...] + b1[...])
    t = _silu(t)
    m = _silu(jnp.dot(t.astype(jnp.bfloat16), W2[...],
                      preferred_element_type=jnp.float32) + b2[...])
    out[...] = _ln(m, g[...], b[...])


def _node_body(xc, fc, p0, p1, g2, b2, gn, bn, n1h, n1m, bn1, Wn2, bn2,
               xout, fout):
    x = xc[...]
    feats = fc[...]
    B = feats.shape[0]
    mi = _ln(p0[...] + p1[...], g2[...], b2[...])
    h = _ln(feats, gn[...], bn[...])
    u = _silu(jnp.dot(h, n1h[...]) + jnp.dot(mi, n1m[...]) + bn1[...])
    hnew = feats + jnp.dot(u, Wn2[...]) + bn2[...]
    fout[...] = hnew
    lane = lax.broadcasted_iota(jnp.int32, (B, XW), 1)
    xout[...] = jnp.where(lane < FL, x, _feats_to_row(hnew, B))


def _post_body(f1, f2, f3, P1, P2, P3, bp1, W2, bp2, W3, bp3, out):
    f = (jnp.dot(f1[...], P1[...]) + jnp.dot(f2[...], P2[...])
         + jnp.dot(f3[...], P3[...]) + bp1[...])
    f = _silu(f)
    f = _silu(jnp.dot(f, W2[...]) + bp2[...])
    out[...] = _silu(jnp.dot(f, W3[...]) + bp3[...])


def _final_body(p0, p1, lg, sl, rg, cl, Wp, TL, TS, TR, TCc, b1,
                W2, b2, W3, b3, W4, b4, out):
    def oh(ref, k):
        ids = ref[...]
        i = lax.broadcasted_iota(jnp.int32, (ids.shape[0], k), 1)
        return (i == ids).astype(jnp.float32)

    z = (jnp.dot(p0[...] + p1[...], Wp[...]) + jnp.dot(oh(lg, 16), TL[...])
         + jnp.dot(oh(sl, 16), TS[...]) + jnp.dot(oh(rg, 8), TR[...])
         + jnp.dot(oh(cl, 8), TCc[...]) + b1[...])
    z = _silu(z)
    z = _silu(jnp.dot(z, W2[...]) + b2[...])
    z = _silu(jnp.dot(z, W3[...]) + b3[...])
    out[...] = jnp.dot(z, W4[...]) + b4[...]


def _tc_call(body, grid, blocked, full, out_blocks, out_shapes):
    full = [a.reshape(1, -1) if a.ndim == 1 else a for a in full]
    in_specs = [pl.BlockSpec(bs, lambda i: (i, 0)) for _, bs in blocked]
    in_specs += [pl.BlockSpec(a.shape, lambda i, _r=len(a.shape): (0,) * _r)
                 for a in full]
    return pl.pallas_call(
        body,
        grid=(grid,),
        in_specs=in_specs,
        out_specs=[pl.BlockSpec(ob, lambda i: (i, 0)) for ob in out_blocks],
        out_shape=out_shapes,
    )(*[a for a, _ in blocked], *full)


def _sc_gather(xcat, src, dst, E):
    CH2 = _divisor_block(E // NW // 2, 400)
    NIT2 = E // NW // CH2
    mesh = plsc.VectorSubcoreMesh(core_axis_name="c", subcore_axis_name="s")

    @functools.partial(
        pl.kernel, mesh=mesh,
        out_type=(jax.ShapeDtypeStruct((E, XW), jnp.float32),
                  jax.ShapeDtypeStruct((E, XW), jnp.float32)),
        scratch_types=[pltpu.VMEM((CH2,), jnp.int32),
                       pltpu.VMEM((CH2,), jnp.int32),
                       pltpu.VMEM((CH2, XW), jnp.float32),
                       pltpu.VMEM((CH2, XW), jnp.float32),
                       pltpu.SemaphoreType.DMA,
                       pltpu.SemaphoreType.DMA],
    )
    def k(x_hbm, s_hbm, d_hbm, gs_hbm, gd_hbm, idx_s, idx_d, rows_s, rows_d,
          sem_s, sem_d):
        wid = lax.axis_index("s") * NC + lax.axis_index("c")
        base = wid * (E // NW)

        def step(i, _):
            off = base + i * CH2
            pltpu.sync_copy(s_hbm.at[pl.ds(off, CH2)], idx_s)
            cp_s = pltpu.async_copy(x_hbm.at[idx_s], rows_s, sem_s)
            pltpu.sync_copy(d_hbm.at[pl.ds(off, CH2)], idx_d)
            cp_d = pltpu.async_copy(x_hbm.at[idx_d], rows_d, sem_d)
            cp_s.wait()
            pltpu.sync_copy(rows_s, gs_hbm.at[pl.ds(off, CH2)])
            cp_d.wait()
            pltpu.sync_copy(rows_d, gd_hbm.at[pl.ds(off, CH2)])
            return _

        lax.fori_loop(0, NIT2, step, None)

    return k(xcat, src, dst)


def _sc_scatter(vals, idx, zeros, n_rows, width, max_ch):
    R = vals.shape[0]
    PW = R // NW
    CH = _divisor_block(PW, max_ch)
    NIT = PW // CH
    STR = n_rows // NS
    mesh = plsc.VectorSubcoreMesh(core_axis_name="c", subcore_axis_name="s")

    @functools.partial(
        pl.kernel, mesh=mesh,
        out_type=jax.ShapeDtypeStruct((NC, n_rows, width), jnp.float32),
        scratch_types=[pltpu.VMEM_SHARED((n_rows, width), jnp.float32),
                       pltpu.VMEM((CH,), jnp.int32),
                       pltpu.VMEM((CH, width), jnp.float32)],
        compiler_params=pltpu.CompilerParams(use_tc_tiling_on_sc=False),
    )
    def k(v_hbm, i_hbm, z_hbm, out_hbm, accum, idx_v, rows_v):
        c = lax.axis_index("c")
        s = lax.axis_index("s")
        wid = s * NC + c
        pltpu.sync_copy(z_hbm.at[pl.ds(s * STR, STR)],
                        accum.at[pl.ds(s * STR, STR)])
        plsc.subcore_barrier()
        base = wid * PW

        def step(i, _):
            off = base + i * CH
            pltpu.sync_copy(i_hbm.at[pl.ds(off, CH)], idx_v)
            pltpu.sync_copy(v_hbm.at[pl.ds(off, CH)], rows_v)
            pltpu.sync_copy(rows_v, accum.at[idx_v], add=True)
            return _

        lax.fori_loop(0, NIT, step, None)
        plsc.subcore_barrier()
        pltpu.sync_copy(accum.at[pl.ds(s * STR, STR)],
                        out_hbm.at[c, pl.ds(s * STR, STR)])

    return k(vals, idx, zeros)


def kernel(params, charges, crds_3d, atom_id, ring_id, hybr_id, arom_id,
           edge_index, batch, lgnd_id, slvn_id, rgnt_id, clst_id):
    N = charges.shape[0]
    E = edge_index.shape[1]
    G = lgnd_id.shape[0]
    NP = -(-N // (NW * 8)) * (NW * 8)
    EP = -(-E // (NW * 1600)) * (NW * 1600)
    NSEG = -(-(G + 1) // 128) * 128
    BN = _divisor_block(NP, 2048)
    BE = _divisor_block(EP, 4096)
    f32 = jnp.float32
    bf16 = jnp.bfloat16

    p = params

    def padi(x):
        return jnp.pad(x.astype(jnp.int32), (0, NP - N)).reshape(NP, 1)

    nfeat = jnp.zeros((NP, 8), f32)
    nfeat = nfeat.at[:N, 0:3].set(crds_3d).at[:N, 3].set(charges[:, 0])
    atom_p, ring_p, hybr_p, arom_p = map(padi, (atom_id, ring_id, hybr_id, arom_id))
    src = jnp.pad(edge_index[0].astype(jnp.int32), (0, EP - E))
    dst = jnp.pad(edge_index[1].astype(jnp.int32), (0, EP - E),
                  constant_values=N)
    batch_p = jnp.pad(batch.astype(jnp.int32), (0, NP - N), constant_values=G)
    z16 = jnp.zeros((NP, MD), f32)
    z128 = jnp.zeros((NSEG, 2 * FD), f32)

    w1 = p["pre1"]["w"]
    Ta = _pad_to(p["atom_em"] @ w1[0:64], (16, 128))
    Tr = _pad_to(p["ring_em"] @ w1[64:128], (8, 128))
    Th = _pad_to(p["hybr_em"] @ w1[128:192], (8, 128))
    Tar = _pad_to(p["arom_em"] @ w1[192:256], (8, 128))
    cw = p["chrg"]["w"] @ w1[256:320]
    Wc = jnp.zeros((8, 128), f32).at[3].set(cw[0])
    b0 = p["pre1"]["b"] + p["chrg"]["b"] @ w1[256:320]

    xcat, feats = _tc_call(
        _embed_body, NP // BN,
        [(atom_p, (BN, 1)), (ring_p, (BN, 1)), (hybr_p, (BN, 1)),
         (arom_p, (BN, 1)), (nfeat, (BN, 8))],
        [Ta, Tr, Th, Tar, Wc, b0, p["pre2"]["w"], p["pre2"]["b"],
         p["pre3"]["w"], p["pre3"]["b"]],
        [(BN, XW), (BN, FD)],
        [jax.ShapeDtypeStruct((NP, XW), f32),
         jax.ShapeDtypeStruct((NP, FD), f32)])

    feats_list = []
    for kp in p["kernels"]:
        e1w, e1b = kp["e1"]["w"], kp["e1"]["b"]
        W1d = jnp.zeros((XW, H1), f32).at[FL:FL + FD, 0:258].set(e1w[0:64])
        W1s = jnp.zeros((XW, H1), f32).at[FL:FL + FD, 0:258].set(e1w[64:128])
        wdv = _pad_to(e1w[128:129], (1, H1))
        b1 = _pad_to(e1b, (H1,))
        W2 = _pad_to(kp["e2"]["w"], (H1, MD))

        gs, gd = _sc_gather(xcat, src, dst, EP)
        m = _tc_call(
            _edge_body, EP // BE,
            [(gd, (BE, XW)), (gs, (BE, XW))],
            [W1d.astype(bf16), W1s.astype(bf16), wdv, b1,
             W2.astype(bf16), kp["e2"]["b"], kp["en1_g"], kp["en1_b"]],
            [(BE, MD)], [jax.ShapeDtypeStruct((EP, MD), f32)])[0]

        parts = _sc_scatter(m, dst, z16, NP, MD, 800)
        xcat, feats = _tc_call(
            _node_body, NP // BN,
            [(xcat, (BN, XW)), (feats, (BN, FD)),
             (parts[0], (BN, MD)), (parts[1], (BN, MD))],
            [kp["en2_g"], kp["en2_b"], kp["nn1_g"], kp["nn1_b"],
             kp["n1"]["w"][0:FD], kp["n1"]["w"][FD:FD + MD], kp["n1"]["b"],
             kp["n2"]["w"], kp["n2"]["b"]],
            [(BN, XW), (BN, FD)],
            [jax.ShapeDtypeStruct((NP, XW), f32),
             jax.ShapeDtypeStruct((NP, FD), f32)])
        feats_list.append(feats)

    pw = p["post1"]["w"]
    f = _tc_call(
        _post_body, NP // BN,
        [(feats_list[0], (BN, FD)), (feats_list[1], (BN, FD)),
         (feats_list[2], (BN, FD))],
        [pw[0:64], pw[64:128], pw[128:192], p["post1"]["b"],
         p["post2"]["w"], p["post2"]["b"], p["post3"]["w"], p["post3"]["b"]],
        [(BN, 128)], [jax.ShapeDtypeStruct((NP, 128), f32)])[0]

    pooled = _sc_scatter(f, batch_p, z128, NSEG, 2 * FD, 784)

    wp1 = p["pp1"]["w"]
    TL = _pad_to(p["lig_emb"] @ wp1[128:192], (16, 512))
    TS = _pad_to(p["sol_emb"] @ wp1[192:256], (16, 512))
    TR = _pad_to(p["rgn_emb"] @ wp1[256:320], (8, 512))
    TCc = _pad_to(p["cat_emb"] @ wp1[320:384], (8, 512))

    def padg(x):
        return jnp.pad(x.astype(jnp.int32), (0, NSEG - G)).reshape(NSEG, 1)

    out = _tc_call(
        _final_body, 1,
        [(pooled[0], (NSEG, 128)), (pooled[1], (NSEG, 128)),
         (padg(lgnd_id), (NSEG, 1)), (padg(slvn_id), (NSEG, 1)),
         (padg(rgnt_id), (NSEG, 1)), (padg(clst_id), (NSEG, 1))],
        [wp1[0:128], TL, TS, TR, TCc, p["pp1"]["b"], p["pp2"]["w"],
         p["pp2"]["b"], p["pp3"]["w"], p["pp3"]["b"], p["pp4"]["w"],
         p["pp4"]["b"]],
        [(NSEG, 1)], [jax.ShapeDtypeStruct((NSEG, 1), f32)])[0]
    return out[:G, 0]


# speedup vs baseline: 2.2896x; 1.0727x over previous
import functools

import jax
import jax.numpy as jnp
from jax import lax
from jax.experimental import pallas as pl
from jax.experimental.pallas import tpu as pltpu
from jax.experimental.pallas import tpu_sc as plsc

XW = 128
FL = 8
FD = 64
MD = 16
H1 = 384
NW = 32
NC = 2
NS = 16


def _divisor_block(n, max_b, mult=8):
    best = None
    for b in range(mult, max_b + 1, mult):
        if n % b == 0:
            best = b
    if best is None:
        raise ValueError(f"no block for n={n} max={max_b}")
    return best


def _silu(x):
    return x * jax.nn.sigmoid(x)


def _ln(x, g, b, eps=1e-5):
    m = jnp.mean(x, axis=-1, keepdims=True)
    v = jnp.mean((x - m) * (x - m), axis=-1, keepdims=True)
    return (x - m) * jax.lax.rsqrt(v + eps) * g + b


def _pad_to(x, shape):
    pads = [(0, t - s) for s, t in zip(x.shape, shape)]
    return jnp.pad(x, pads)


def _coords_to_row(c, n_rows):
    return jnp.concatenate(
        [c, jnp.zeros((n_rows, XW - 3), jnp.float32)], axis=1)


def _feats_to_row(h, n_rows):
    return jnp.concatenate(
        [jnp.zeros((n_rows, FL), jnp.float32), h,
         jnp.zeros((n_rows, XW - FL - FD), jnp.float32)], axis=1)


def _embed_body(atom, ring, hybr, arom, nfeat, Ta, Tr, Th, Tar, Wc, b0,
                W2, b2, W3, b3, xout, fout):
    def oh(ref, k):
        ids = ref[...]
        i = lax.broadcasted_iota(jnp.int32, (ids.shape[0], k), 1)
        return (i == ids).astype(jnp.float32)

    nf = nfeat[...]
    B = nf.shape[0]
    h = (jnp.dot(oh(atom, 16), Ta[...]) + jnp.dot(oh(ring, 8), Tr[...])
         + jnp.dot(oh(hybr, 8), Th[...]) + jnp.dot(oh(arom, 8), Tar[...])
         + jnp.dot(nf, Wc[...]) + b0[...])
    h = _silu(h)
    h = _silu(jnp.dot(h, W2[...]) + b2[...])
    h = _silu(jnp.dot(h, W3[...]) + b3[...])
    fout[...] = h
    xout[...] = _coords_to_row(nf[:, 0:3], B) + _feats_to_row(h, B)


def _edge_body(gd, gs, W1d, W1s, wdv, b1, W2, b2, g, b, out):
    gdv = gd[...]
    gsv = gs[...]
    B = gdv.shape[0]
    d = gsv[:, 0:3] - gdv[:, 0:3]
    rd = jnp.sum(d * d, axis=1, keepdims=True)
    t = (jnp.dot(gdv.astype(jnp.bfloat16), W1d[...],
                 preferred_element_type=jnp.float32)
         + jnp.dot(gsv.astype(jnp.bfloat16), W1s[...],
                   preferred_element_type=jnp.float32)
         + rd * wdv[...] + b1[...])
    t = _silu(t)
    m = _silu(jnp.dot(t.astype(jnp.bfloat16), W2[...],
                      preferred_element_type=jnp.float32) + b2[...])
    out[...] = _ln(m, g[...], b[...])


def _node_body(xc, fc, p0, p1, g2, b2, gn, bn, n1h, n1m, bn1, Wn2, bn2,
               xout, fout):
    x = xc[...]
    feats = fc[...]
    B = feats.shape[0]
    mi = _ln(p0[...] + p1[...], g2[...], b2[...])
    h = _ln(feats, gn[...], bn[...])
    u = _silu(jnp.dot(h, n1h[...]) + jnp.dot(mi, n1m[...]) + bn1[...])
    hnew = feats + jnp.dot(u, Wn2[...]) + bn2[...]
    fout[...] = hnew
    lane = lax.broadcasted_iota(jnp.int32, (B, XW), 1)
    xout[...] = jnp.where(lane < FL, x, _feats_to_row(hnew, B))


def _post_body(f1, f2, f3, P1, P2, P3, bp1, W2, bp2, W3, bp3, out):
    f = (jnp.dot(f1[...], P1[...]) + jnp.dot(f2[...], P2[...])
         + jnp.dot(f3[...], P3[...]) + bp1[...])
    f = _silu(f)
    f = _silu(jnp.dot(f, W2[...]) + bp2[...])
    out[...] = _silu(jnp.dot(f, W3[...]) + bp3[...])


def _final_body(p0, p1, lg, sl, rg, cl, Wp, TL, TS, TR, TCc, b1,
                W2, b2, W3, b3, W4, b4, out):
    def oh(ref, k):
        ids = ref[...]
        i = lax.broadcasted_iota(jnp.int32, (ids.shape[0], k), 1)
        return (i == ids).astype(jnp.float32)

    z = (jnp.dot(p0[...] + p1[...], Wp[...]) + jnp.dot(oh(lg, 16), TL[...])
         + jnp.dot(oh(sl, 16), TS[...]) + jnp.dot(oh(rg, 8), TR[...])
         + jnp.dot(oh(cl, 8), TCc[...]) + b1[...])
    z = _silu(z)
    z = _silu(jnp.dot(z, W2[...]) + b2[...])
    z = _silu(jnp.dot(z, W3[...]) + b3[...])
    out[...] = jnp.dot(z, W4[...]) + b4[...]


def _tc_call(body, grid, blocked, full, out_blocks, out_shapes):
    full = [a.reshape(1, -1) if a.ndim == 1 else a for a in full]
    in_specs = [pl.BlockSpec(bs, lambda i: (i, 0)) for _, bs in blocked]
    in_specs += [pl.BlockSpec(a.shape, lambda i, _r=len(a.shape): (0,) * _r)
                 for a in full]
    return pl.pallas_call(
        body,
        grid=(grid,),
        in_specs=in_specs,
        out_specs=[pl.BlockSpec(ob, lambda i: (i, 0)) for ob in out_blocks],
        out_shape=out_shapes,
    )(*[a for a, _ in blocked], *full)


def _sc_gather(xcat, src, dst, E):
    PW = E // NW
    CH = _divisor_block(PW // 2, 128)
    NIT = PW // CH
    mesh = plsc.VectorSubcoreMesh(core_axis_name="c", subcore_axis_name="s")

    @functools.partial(
        pl.kernel, mesh=mesh,
        out_type=(jax.ShapeDtypeStruct((E, XW), jnp.float32),
                  jax.ShapeDtypeStruct((E, XW), jnp.float32)),
        scratch_types=[pltpu.VMEM((PW,), jnp.int32),
                       pltpu.VMEM((PW,), jnp.int32),
                       [pltpu.VMEM((CH, XW), jnp.float32) for _ in range(2)],
                       [pltpu.VMEM((CH, XW), jnp.float32) for _ in range(2)],
                       [pltpu.SemaphoreType.DMA for _ in range(4)],
                       [pltpu.SemaphoreType.DMA for _ in range(4)]],
    )
    def k(x_hbm, s_hbm, d_hbm, gs_hbm, gd_hbm, idx_s, idx_d, rows_s, rows_d,
          gsem, wsem):
        wid = lax.axis_index("s") * NC + lax.axis_index("c")
        base = wid * PW
        pltpu.sync_copy(s_hbm.at[pl.ds(base, PW)], idx_s)
        pltpu.sync_copy(d_hbm.at[pl.ds(base, PW)], idx_d)

        def gath(i, b):
            sl = pl.ds(i * CH, CH)
            pltpu.async_copy(x_hbm.at[idx_s.at[sl]], rows_s[b], gsem[b])
            pltpu.async_copy(x_hbm.at[idx_d.at[sl]], rows_d[b], gsem[2 + b])

        gath(0, 0)
        gath(1, 1)

        def step(j, _):
            i = 2 * j
            for b in (0, 1):
                ib = i + b
                sl = pl.ds(base + ib * CH, CH)
                pltpu.make_async_copy(x_hbm.at[idx_s.at[pl.ds(0, CH)]],
                                      rows_s[b], gsem[b]).wait()
                pltpu.async_copy(rows_s[b], gs_hbm.at[sl], wsem[b])
                pltpu.make_async_copy(x_hbm.at[idx_d.at[pl.ds(0, CH)]],
                                      rows_d[b], gsem[2 + b]).wait()
                pltpu.async_copy(rows_d[b], gd_hbm.at[sl], wsem[2 + b])

                @pl.when(ib + 2 < NIT)
                def _():
                    pltpu.make_async_copy(rows_s[b], gs_hbm.at[sl],
                                          wsem[b]).wait()
                    pltpu.make_async_copy(rows_d[b], gd_hbm.at[sl],
                                          wsem[2 + b]).wait()
                    gath(ib + 2, b)
            return _

        lax.fori_loop(0, NIT // 2, step, None)
        for b in (0, 1):
            sl = pl.ds(base, CH)
            pltpu.make_async_copy(rows_s[b], gs_hbm.at[sl], wsem[b]).wait()
            pltpu.make_async_copy(rows_d[b], gd_hbm.at[sl], wsem[2 + b]).wait()

    return k(xcat, src, dst)


def _sc_scatter(vals, idx, zeros, n_rows, width, max_ch):
    R = vals.shape[0]
    PW = R // NW
    CH = _divisor_block(PW, max_ch)
    NIT = PW // CH
    STR = n_rows // NS
    mesh = plsc.VectorSubcoreMesh(core_axis_name="c", subcore_axis_name="s")

    @functools.partial(
        pl.kernel, mesh=mesh,
        out_type=jax.ShapeDtypeStruct((NC, n_rows, width), jnp.float32),
        scratch_types=[pltpu.VMEM_SHARED((n_rows, width), jnp.float32),
                       pltpu.VMEM((CH,), jnp.int32),
                       pltpu.VMEM((CH, width), jnp.float32)],
        compiler_params=pltpu.CompilerParams(use_tc_tiling_on_sc=False),
    )
    def k(v_hbm, i_hbm, z_hbm, out_hbm, accum, idx_v, rows_v):
        c = lax.axis_index("c")
        s = lax.axis_index("s")
        wid = s * NC + c
        pltpu.sync_copy(z_hbm.at[pl.ds(s * STR, STR)],
                        accum.at[pl.ds(s * STR, STR)])
        plsc.subcore_barrier()
        base = wid * PW

        def step(i, _):
            off = base + i * CH
            pltpu.sync_copy(i_hbm.at[pl.ds(off, CH)], idx_v)
            pltpu.sync_copy(v_hbm.at[pl.ds(off, CH)], rows_v)
            pltpu.sync_copy(rows_v, accum.at[idx_v], add=True)
            return _

        lax.fori_loop(0, NIT, step, None)
        plsc.subcore_barrier()
        pltpu.sync_copy(accum.at[pl.ds(s * STR, STR)],
                        out_hbm.at[c, pl.ds(s * STR, STR)])

    return k(vals, idx, zeros)


def kernel(params, charges, crds_3d, atom_id, ring_id, hybr_id, arom_id,
           edge_index, batch, lgnd_id, slvn_id, rgnt_id, clst_id):
    N = charges.shape[0]
    E = edge_index.shape[1]
    G = lgnd_id.shape[0]
    NP = -(-N // (NW * 8)) * (NW * 8)
    EP = -(-E // (NW * 1600)) * (NW * 1600)
    NSEG = -(-(G + 1) // 128) * 128
    BN = _divisor_block(NP, 2048)
    BE = _divisor_block(EP, 4096)
    f32 = jnp.float32
    bf16 = jnp.bfloat16

    p = params

    def padi(x):
        return jnp.pad(x.astype(jnp.int32), (0, NP - N)).reshape(NP, 1)

    nfeat = jnp.zeros((NP, 8), f32)
    nfeat = nfeat.at[:N, 0:3].set(crds_3d).at[:N, 3].set(charges[:, 0])
    atom_p, ring_p, hybr_p, arom_p = map(padi, (atom_id, ring_id, hybr_id, arom_id))
    src = jnp.pad(edge_index[0].astype(jnp.int32), (0, EP - E))
    dst = jnp.pad(edge_index[1].astype(jnp.int32), (0, EP - E),
                  constant_values=N)
    batch_p = jnp.pad(batch.astype(jnp.int32), (0, NP - N), constant_values=G)
    z16 = jnp.zeros((NP, MD), f32)
    z128 = jnp.zeros((NSEG, 2 * FD), f32)

    w1 = p["pre1"]["w"]
    Ta = _pad_to(p["atom_em"] @ w1[0:64], (16, 128))
    Tr = _pad_to(p["ring_em"] @ w1[64:128], (8, 128))
    Th = _pad_to(p["hybr_em"] @ w1[128:192], (8, 128))
    Tar = _pad_to(p["arom_em"] @ w1[192:256], (8, 128))
    cw = p["chrg"]["w"] @ w1[256:320]
    Wc = jnp.zeros((8, 128), f32).at[3].set(cw[0])
    b0 = p["pre1"]["b"] + p["chrg"]["b"] @ w1[256:320]

    xcat, feats = _tc_call(
        _embed_body, NP // BN,
        [(atom_p, (BN, 1)), (ring_p, (BN, 1)), (hybr_p, (BN, 1)),
         (arom_p, (BN, 1)), (nfeat, (BN, 8))],
        [Ta, Tr, Th, Tar, Wc, b0, p["pre2"]["w"], p["pre2"]["b"],
         p["pre3"]["w"], p["pre3"]["b"]],
        [(BN, XW), (BN, FD)],
        [jax.ShapeDtypeStruct((NP, XW), f32),
         jax.ShapeDtypeStruct((NP, FD), f32)])

    feats_list = []
    for kp in p["kernels"]:
        e1w, e1b = kp["e1"]["w"], kp["e1"]["b"]
        W1d = jnp.zeros((XW, H1), f32).at[FL:FL + FD, 0:258].set(e1w[0:64])
        W1s = jnp.zeros((XW, H1), f32).at[FL:FL + FD, 0:258].set(e1w[64:128])
        wdv = _pad_to(e1w[128:129], (1, H1))
        b1 = _pad_to(e1b, (H1,))
        W2 = _pad_to(kp["e2"]["w"], (H1, MD))

        gs, gd = _sc_gather(xcat, src, dst, EP)
        m = _tc_call(
            _edge_body, EP // BE,
            [(gd, (BE, XW)), (gs, (BE, XW))],
            [W1d.astype(bf16), W1s.astype(bf16), wdv, b1,
             W2.astype(bf16), kp["e2"]["b"], kp["en1_g"], kp["en1_b"]],
            [(BE, MD)], [jax.ShapeDtypeStruct((EP, MD), f32)])[0]

        parts = _sc_scatter(m, dst, z16, NP, MD, 800)
        xcat, feats = _tc_call(
            _node_body, NP // BN,
            [(xcat, (BN, XW)), (feats, (BN, FD)),
             (parts[0], (BN, MD)), (parts[1], (BN, MD))],
            [kp["en2_g"], kp["en2_b"], kp["nn1_g"], kp["nn1_b"],
             kp["n1"]["w"][0:FD], kp["n1"]["w"][FD:FD + MD], kp["n1"]["b"],
             kp["n2"]["w"], kp["n2"]["b"]],
            [(BN, XW), (BN, FD)],
            [jax.ShapeDtypeStruct((NP, XW), f32),
             jax.ShapeDtypeStruct((NP, FD), f32)])
        feats_list.append(feats)

    pw = p["post1"]["w"]
    f = _tc_call(
        _post_body, NP // BN,
        [(feats_list[0], (BN, FD)), (feats_list[1], (BN, FD)),
         (feats_list[2], (BN, FD))],
        [pw[0:64], pw[64:128], pw[128:192], p["post1"]["b"],
         p["post2"]["w"], p["post2"]["b"], p["post3"]["w"], p["post3"]["b"]],
        [(BN, 128)], [jax.ShapeDtypeStruct((NP, 128), f32)])[0]

    pooled = _sc_scatter(f, batch_p, z128, NSEG, 2 * FD, 784)

    wp1 = p["pp1"]["w"]
    TL = _pad_to(p["lig_emb"] @ wp1[128:192], (16, 512))
    TS = _pad_to(p["sol_emb"] @ wp1[192:256], (16, 512))
    TR = _pad_to(p["rgn_emb"] @ wp1[256:320], (8, 512))
    TCc = _pad_to(p["cat_emb"] @ wp1[320:384], (8, 512))

    def padg(x):
        return jnp.pad(x.astype(jnp.int32), (0, NSEG - G)).reshape(NSEG, 1)

    out = _tc_call(
        _final_body, 1,
        [(pooled[0], (NSEG, 128)), (pooled[1], (NSEG, 128)),
         (padg(lgnd_id), (NSEG, 1)), (padg(slvn_id), (NSEG, 1)),
         (padg(rgnt_id), (NSEG, 1)), (padg(clst_id), (NSEG, 1))],
        [wp1[0:128], TL, TS, TR, TCc, p["pp1"]["b"], p["pp2"]["w"],
         p["pp2"]["b"], p["pp3"]["w"], p["pp3"]["b"], p["pp4"]["w"],
         p["pp4"]["b"]],
        [(NSEG, 1)], [jax.ShapeDtypeStruct((NSEG, 1), f32)])[0]
    return out[:G, 0]


# speedup vs baseline: 2.7189x; 1.1875x over previous
import functools

import jax
import jax.numpy as jnp
from jax import lax
from jax.experimental import pallas as pl
from jax.experimental.pallas import tpu as pltpu
from jax.experimental.pallas import tpu_sc as plsc

XW = 128
FL = 8
FD = 64
MD = 16
H1 = 384
NW = 32
NC = 2
NS = 16


def _divisor_block(n, max_b, mult=8):
    best = None
    for b in range(mult, max_b + 1, mult):
        if n % b == 0:
            best = b
    if best is None:
        raise ValueError(f"no block for n={n} max={max_b}")
    return best


def _silu(x):
    return x * jax.nn.sigmoid(x)


def _ln(x, g, b, eps=1e-5):
    m = jnp.mean(x, axis=-1, keepdims=True)
    v = jnp.mean((x - m) * (x - m), axis=-1, keepdims=True)
    return (x - m) * jax.lax.rsqrt(v + eps) * g + b


def _pad_to(x, shape):
    pads = [(0, t - s) for s, t in zip(x.shape, shape)]
    return jnp.pad(x, pads)


def _coords_to_row(c, n_rows):
    return jnp.concatenate(
        [c, jnp.zeros((n_rows, XW - 3), jnp.float32)], axis=1)


def _feats_to_row(h, n_rows):
    return jnp.concatenate(
        [jnp.zeros((n_rows, FL), jnp.float32), h,
         jnp.zeros((n_rows, XW - FL - FD), jnp.float32)], axis=1)


def _embed_body(atom, ring, hybr, arom, nfeat, Ta, Tr, Th, Tar, Wc, b0,
                W2, b2, W3, b3, xout, fout):
    def oh(ref, k):
        ids = ref[...]
        i = lax.broadcasted_iota(jnp.int32, (ids.shape[0], k), 1)
        return (i == ids).astype(jnp.float32)

    nf = nfeat[...]
    B = nf.shape[0]
    h = (jnp.dot(oh(atom, 16), Ta[...]) + jnp.dot(oh(ring, 8), Tr[...])
         + jnp.dot(oh(hybr, 8), Th[...]) + jnp.dot(oh(arom, 8), Tar[...])
         + jnp.dot(nf, Wc[...]) + b0[...])
    h = _silu(h)
    h = _silu(jnp.dot(h, W2[...]) + b2[...])
    h = _silu(jnp.dot(h, W3[...]) + b3[...])
    fout[...] = h
    xout[...] = _coords_to_row(nf[:, 0:3], B) + _feats_to_row(h, B)


def _edge_body(gd, gs, W1d, W1s, wdv, b1, W2, b2, g, b, out):
    gdv = gd[...]
    gsv = gs[...]
    B = gdv.shape[0]
    d = gsv[:, 0:3] - gdv[:, 0:3]
    rd = jnp.sum(d * d, axis=1, keepdims=True)
    t = (jnp.dot(gdv.astype(jnp.bfloat16), W1d[...],
                 preferred_element_type=jnp.float32)
         + jnp.dot(gsv.astype(jnp.bfloat16), W1s[...],
                   preferred_element_type=jnp.float32)
         + rd * wdv[...] + b1[...])
    t = _silu(t)
    m = _silu(jnp.dot(t.astype(jnp.bfloat16), W2[...],
                      preferred_element_type=jnp.float32) + b2[...])
    out[...] = _ln(m, g[...], b[...])


def _node_body(xc, fc, p0, p1, p2, p3, g2, b2, gn, bn, n1h, n1m, bn1,
               Wn2, bn2, xout, fout):
    x = xc[...]
    feats = fc[...]
    B = feats.shape[0]
    mi = _ln(p0[...] + p1[...] + p2[...] + p3[...], g2[...], b2[...])
    h = _ln(feats, gn[...], bn[...])
    u = _silu(jnp.dot(h, n1h[...]) + jnp.dot(mi, n1m[...]) + bn1[...])
    hnew = feats + jnp.dot(u, Wn2[...]) + bn2[...]
    fout[...] = hnew
    lane = lax.broadcasted_iota(jnp.int32, (B, XW), 1)
    xout[...] = jnp.where(lane < FL, x, _feats_to_row(hnew, B))


def _post_body(f1, f2, f3, P1, P2, P3, bp1, W2, bp2, W3, bp3, out):
    f = (jnp.dot(f1[...], P1[...]) + jnp.dot(f2[...], P2[...])
         + jnp.dot(f3[...], P3[...]) + bp1[...])
    f = _silu(f)
    f = _silu(jnp.dot(f, W2[...]) + bp2[...])
    out[...] = _silu(jnp.dot(f, W3[...]) + bp3[...])


def _final_body(p0, p1, lg, sl, rg, cl, Wp, TL, TS, TR, TCc, b1,
                W2, b2, W3, b3, W4, b4, out):
    def oh(ref, k):
        ids = ref[...]
        i = lax.broadcasted_iota(jnp.int32, (ids.shape[0], k), 1)
        return (i == ids).astype(jnp.float32)

    z = (jnp.dot(p0[...] + p1[...], Wp[...]) + jnp.dot(oh(lg, 16), TL[...])
         + jnp.dot(oh(sl, 16), TS[...]) + jnp.dot(oh(rg, 8), TR[...])
         + jnp.dot(oh(cl, 8), TCc[...]) + b1[...])
    z = _silu(z)
    z = _silu(jnp.dot(z, W2[...]) + b2[...])
    z = _silu(jnp.dot(z, W3[...]) + b3[...])
    out[...] = jnp.dot(z, W4[...]) + b4[...]


def _tc_call(body, grid, blocked, full, out_blocks, out_shapes):
    full = [a.reshape(1, -1) if a.ndim == 1 else a for a in full]
    in_specs = [pl.BlockSpec(bs, lambda i: (i, 0)) for _, bs in blocked]
    in_specs += [pl.BlockSpec(a.shape, lambda i, _r=len(a.shape): (0,) * _r)
                 for a in full]
    return pl.pallas_call(
        body,
        grid=(grid,),
        in_specs=in_specs,
        out_specs=[pl.BlockSpec(ob, lambda i: (i, 0)) for ob in out_blocks],
        out_shape=out_shapes,
    )(*[a for a, _ in blocked], *full)


def _sc_gather(xcat, src, dst, E):
    PW = E // NW
    CH = _divisor_block(PW // 2, 128)
    NIT = PW // CH
    mesh = plsc.VectorSubcoreMesh(core_axis_name="c", subcore_axis_name="s")

    @functools.partial(
        pl.kernel, mesh=mesh,
        out_type=(jax.ShapeDtypeStruct((E, XW), jnp.float32),
                  jax.ShapeDtypeStruct((E, XW), jnp.float32)),
        scratch_types=[pltpu.VMEM((PW,), jnp.int32),
                       pltpu.VMEM((PW,), jnp.int32),
                       [pltpu.VMEM((CH, XW), jnp.float32) for _ in range(2)],
                       [pltpu.VMEM((CH, XW), jnp.float32) for _ in range(2)],
                       [pltpu.SemaphoreType.DMA for _ in range(4)],
                       [pltpu.SemaphoreType.DMA for _ in range(4)]],
    )
    def k(x_hbm, s_hbm, d_hbm, gs_hbm, gd_hbm, idx_s, idx_d, rows_s, rows_d,
          gsem, wsem):
        wid = lax.axis_index("s") * NC + lax.axis_index("c")
        base = wid * PW
        pltpu.sync_copy(s_hbm.at[pl.ds(base, PW)], idx_s)
        pltpu.sync_copy(d_hbm.at[pl.ds(base, PW)], idx_d)

        def gath(i, b):
            sl = pl.ds(i * CH, CH)
            pltpu.async_copy(x_hbm.at[idx_s.at[sl]], rows_s[b], gsem[b])
            pltpu.async_copy(x_hbm.at[idx_d.at[sl]], rows_d[b], gsem[2 + b])

        gath(0, 0)
        gath(1, 1)

        def step(j, _):
            i = 2 * j
            for b in (0, 1):
                ib = i + b
                sl = pl.ds(base + ib * CH, CH)
                pltpu.make_async_copy(x_hbm.at[idx_s.at[pl.ds(0, CH)]],
                                      rows_s[b], gsem[b]).wait()
                pltpu.async_copy(rows_s[b], gs_hbm.at[sl], wsem[b])
                pltpu.make_async_copy(x_hbm.at[idx_d.at[pl.ds(0, CH)]],
                                      rows_d[b], gsem[2 + b]).wait()
                pltpu.async_copy(rows_d[b], gd_hbm.at[sl], wsem[2 + b])

                @pl.when(ib + 2 < NIT)
                def _():
                    pltpu.make_async_copy(rows_s[b], gs_hbm.at[sl],
                                          wsem[b]).wait()
                    pltpu.make_async_copy(rows_d[b], gd_hbm.at[sl],
                                          wsem[2 + b]).wait()
                    gath(ib + 2, b)
            return _

        lax.fori_loop(0, NIT // 2, step, None)
        for b in (0, 1):
            sl = pl.ds(base, CH)
            pltpu.make_async_copy(rows_s[b], gs_hbm.at[sl], wsem[b]).wait()
            pltpu.make_async_copy(rows_d[b], gd_hbm.at[sl], wsem[2 + b]).wait()

    return k(xcat, src, dst)


def _sc_scatter(vals, idx, zeros, n_rows, width, max_ch):
    R = vals.shape[0]
    PW = R // NW
    CH = _divisor_block(PW, max_ch)
    NIT = PW // CH
    STR = n_rows // NS
    mesh = plsc.VectorSubcoreMesh(core_axis_name="c", subcore_axis_name="s")

    @functools.partial(
        pl.kernel, mesh=mesh,
        out_type=jax.ShapeDtypeStruct((NC, n_rows, width), jnp.float32),
        scratch_types=[pltpu.VMEM_SHARED((n_rows, width), jnp.float32),
                       pltpu.VMEM((CH,), jnp.int32),
                       pltpu.VMEM((CH, width), jnp.float32)],
        compiler_params=pltpu.CompilerParams(use_tc_tiling_on_sc=False),
    )
    def k(v_hbm, i_hbm, z_hbm, out_hbm, accum, idx_v, rows_v):
        c = lax.axis_index("c")
        s = lax.axis_index("s")
        wid = s * NC + c
        pltpu.sync_copy(z_hbm.at[pl.ds(s * STR, STR)],
                        accum.at[pl.ds(s * STR, STR)])
        plsc.subcore_barrier()
        base = wid * PW

        def step(i, _):
            off = base + i * CH
            pltpu.sync_copy(i_hbm.at[pl.ds(off, CH)], idx_v)
            pltpu.sync_copy(v_hbm.at[pl.ds(off, CH)], rows_v)
            pltpu.sync_copy(rows_v, accum.at[idx_v], add=True)
            return _

        lax.fori_loop(0, NIT, step, None)
        plsc.subcore_barrier()
        pltpu.sync_copy(accum.at[pl.ds(s * STR, STR)],
                        out_hbm.at[c, pl.ds(s * STR, STR)])

    return k(vals, idx, zeros)


def kernel(params, charges, crds_3d, atom_id, ring_id, hybr_id, arom_id,
           edge_index, batch, lgnd_id, slvn_id, rgnt_id, clst_id):
    N = charges.shape[0]
    E = edge_index.shape[1]
    G = lgnd_id.shape[0]
    NP = -(-N // (NW * 8)) * (NW * 8)
    EP = -(-E // (NW * 1600)) * (NW * 1600)
    EH = EP // 2
    NSEG = -(-(G + 1) // 128) * 128
    BN = _divisor_block(NP, 2048)
    BE = _divisor_block(EH, 4096)
    f32 = jnp.float32
    bf16 = jnp.bfloat16

    p = params

    def padi(x):
        return jnp.pad(x.astype(jnp.int32), (0, NP - N)).reshape(NP, 1)

    nfeat = jnp.zeros((NP, 8), f32)
    nfeat = nfeat.at[:N, 0:3].set(crds_3d).at[:N, 3].set(charges[:, 0])
    atom_p, ring_p, hybr_p, arom_p = map(padi, (atom_id, ring_id, hybr_id, arom_id))
    src = jnp.pad(edge_index[0].astype(jnp.int32), (0, EP - E))
    dst = jnp.pad(edge_index[1].astype(jnp.int32), (0, EP - E),
                  constant_values=N)
    src_h = (src[:EH], src[EH:])
    dst_h = (dst[:EH], dst[EH:])
    batch_p = jnp.pad(batch.astype(jnp.int32), (0, NP - N), constant_values=G)
    z16 = jnp.zeros((NP, MD), f32)
    z128 = jnp.zeros((NSEG, 2 * FD), f32)

    w1 = p["pre1"]["w"]
    Ta = _pad_to(p["atom_em"] @ w1[0:64], (16, 128))
    Tr = _pad_to(p["ring_em"] @ w1[64:128], (8, 128))
    Th = _pad_to(p["hybr_em"] @ w1[128:192], (8, 128))
    Tar = _pad_to(p["arom_em"] @ w1[192:256], (8, 128))
    cw = p["chrg"]["w"] @ w1[256:320]
    Wc = jnp.zeros((8, 128), f32).at[3].set(cw[0])
    b0 = p["pre1"]["b"] + p["chrg"]["b"] @ w1[256:320]

    xcat, feats = _tc_call(
        _embed_body, NP // BN,
        [(atom_p, (BN, 1)), (ring_p, (BN, 1)), (hybr_p, (BN, 1)),
         (arom_p, (BN, 1)), (nfeat, (BN, 8))],
        [Ta, Tr, Th, Tar, Wc, b0, p["pre2"]["w"], p["pre2"]["b"],
         p["pre3"]["w"], p["pre3"]["b"]],
        [(BN, XW), (BN, FD)],
        [jax.ShapeDtypeStruct((NP, XW), f32),
         jax.ShapeDtypeStruct((NP, FD), f32)])

    feats_list = []
    for kp in p["kernels"]:
        e1w, e1b = kp["e1"]["w"], kp["e1"]["b"]
        W1d = jnp.zeros((XW, H1), f32).at[FL:FL + FD, 0:258].set(e1w[0:64])
        W1s = jnp.zeros((XW, H1), f32).at[FL:FL + FD, 0:258].set(e1w[64:128])
        wdv = _pad_to(e1w[128:129], (1, H1))
        b1 = _pad_to(e1b, (H1,))
        W2 = _pad_to(kp["e2"]["w"], (H1, MD))

        gath = [_sc_gather(xcat, src_h[h], dst_h[h], EH) for h in (0, 1)]
        ms = [_tc_call(
            _edge_body, EH // BE,
            [(gath[h][1], (BE, XW)), (gath[h][0], (BE, XW))],
            [W1d.astype(bf16), W1s.astype(bf16), wdv, b1,
             W2.astype(bf16), kp["e2"]["b"], kp["en1_g"], kp["en1_b"]],
            [(BE, MD)], [jax.ShapeDtypeStruct((EH, MD), f32)])[0]
            for h in (0, 1)]

        parts = [_sc_scatter(ms[h], dst_h[h], z16, NP, MD, 800)
                 for h in (0, 1)]
        xcat, feats = _tc_call(
            _node_body, NP // BN,
            [(xcat, (BN, XW)), (feats, (BN, FD)),
             (parts[0][0], (BN, MD)), (parts[0][1], (BN, MD)),
             (parts[1][0], (BN, MD)), (parts[1][1], (BN, MD))],
            [kp["en2_g"], kp["en2_b"], kp["nn1_g"], kp["nn1_b"],
             kp["n1"]["w"][0:FD], kp["n1"]["w"][FD:FD + MD], kp["n1"]["b"],
             kp["n2"]["w"], kp["n2"]["b"]],
            [(BN, XW), (BN, FD)],
            [jax.ShapeDtypeStruct((NP, XW), f32),
             jax.ShapeDtypeStruct((NP, FD), f32)])
        feats_list.append(feats)

    pw = p["post1"]["w"]
    f = _tc_call(
        _post_body, NP // BN,
        [(feats_list[0], (BN, FD)), (feats_list[1], (BN, FD)),
         (feats_list[2], (BN, FD))],
        [pw[0:64], pw[64:128], pw[128:192], p["post1"]["b"],
         p["post2"]["w"], p["post2"]["b"], p["post3"]["w"], p["post3"]["b"]],
        [(BN, 128)], [jax.ShapeDtypeStruct((NP, 128), f32)])[0]

    pooled = _sc_scatter(f, batch_p, z128, NSEG, 2 * FD, 784)

    wp1 = p["pp1"]["w"]
    TL = _pad_to(p["lig_emb"] @ wp1[128:192], (16, 512))
    TS = _pad_to(p["sol_emb"] @ wp1[192:256], (16, 512))
    TR = _pad_to(p["rgn_emb"] @ wp1[256:320], (8, 512))
    TCc = _pad_to(p["cat_emb"] @ wp1[320:384], (8, 512))

    def padg(x):
        return jnp.pad(x.astype(jnp.int32), (0, NSEG - G)).reshape(NSEG, 1)

    out = _tc_call(
        _final_body, 1,
        [(pooled[0], (NSEG, 128)), (pooled[1], (NSEG, 128)),
         (padg(lgnd_id), (NSEG, 1)), (padg(slvn_id), (NSEG, 1)),
         (padg(rgnt_id), (NSEG, 1)), (padg(clst_id), (NSEG, 1))],
        [wp1[0:128], TL, TS, TR, TCc, p["pp1"]["b"], p["pp2"]["w"],
         p["pp2"]["b"], p["pp3"]["w"], p["pp3"]["b"], p["pp4"]["w"],
         p["pp4"]["b"]],
        [(NSEG, 1)], [jax.ShapeDtypeStruct((NSEG, 1), f32)])[0]
    return out[:G, 0]
